# baseline (speedup 1.0000x reference)
import functools

import jax
import jax.numpy as jnp
from jax import lax
from jax.experimental import pallas as pl
from jax.experimental.pallas import tpu as pltpu
from jax.experimental.pallas import tpu_sc as plsc

N_CODES = 1024
DIM = 256
N_STAGES = 4
BETA_C = 0.25
BL = 512
ROWS = 8192


def _vq_body(nsteps, z_ref, w_ref, zq_ref, idx_ref, loss_ref, ppl_ref,
             cnt_scr):
    i = pl.program_id(0)

    @pl.when(i == 0)
    def _init():
        loss_ref[...] = jnp.zeros_like(loss_ref)
        cnt_scr[...] = jnp.zeros_like(cnt_scr)

    w = w_ref[...]
    wsq = lax.dot_general(jnp.ones((1, DIM), jnp.float32), w * w,
                          (((1,), (1,)), ((), ())),
                          preferred_element_type=jnp.float32,
                          precision=lax.Precision.HIGHEST)
    w1 = w.astype(jnp.bfloat16)
    w2 = (w - w1.astype(jnp.float32)).astype(jnp.bfloat16)
    w3 = (w - w1.astype(jnp.float32) - w2.astype(jnp.float32)).astype(jnp.bfloat16)

    residual = z_ref[...]
    qsum = jnp.zeros((BL, DIM), jnp.float32)
    lsum = jnp.zeros((1, 1), jnp.float32)
    cnt = jnp.zeros((1, N_CODES), jnp.float32)
    idx_cols = []
    iota = lax.broadcasted_iota(jnp.int32, (BL, N_CODES), 1)
    for q in range(N_STAGES):
        s = lax.dot_general(residual, w, (((1,), (1,)), ((), ())),
                            preferred_element_type=jnp.float32)
        rsq = jnp.sum(residual * residual, axis=1, keepdims=True)
        dist = (rsq + wsq) - 2.0 * s
        dmin = jnp.min(dist, axis=1, keepdims=True)
        idxm = jnp.min(jnp.where(dist <= dmin, iota, N_CODES), axis=1,
                       keepdims=True)
        oh = (iota == idxm).astype(jnp.float32)
        cnt = cnt + jnp.sum(oh, axis=0, keepdims=True)
        ohb = oh.astype(jnp.bfloat16)
        zqd = (lax.dot_general(ohb, w1, (((1,), (0,)), ((), ())),
                               preferred_element_type=jnp.float32)
               + lax.dot_general(ohb, w2, (((1,), (0,)), ((), ())),
                                 preferred_element_type=jnp.float32)
               + lax.dot_general(ohb, w3, (((1,), (0,)), ((), ())),
                                 preferred_element_type=jnp.float32))
        qsum = qsum + zqd
        residual = residual - zqd
        r2 = jnp.sum(residual * residual, axis=1, keepdims=True)
        lsum = lsum + jnp.sum(r2, axis=0, keepdims=True)
        idx_cols.append(idxm)

    zq_ref[...] = qsum
    idx_ref[...] = jnp.concatenate(idx_cols, axis=1)
    loss_ref[...] += lsum
    cnt_scr[...] += cnt

    @pl.when(i == nsteps - 1)
    def _fini():
        loss_ref[...] = loss_ref[...] * (BETA_C / (nsteps * BL * DIM))
        avg = cnt_scr[...] * (1.0 / (nsteps * BL * N_STAGES))
        ent = jnp.sum(avg * jnp.log(avg + 1e-10), axis=1, keepdims=True)
        ppl_ref[...] = jnp.exp(-ent)


def _tc_call(z_flat, weight):
    nsteps = ROWS // BL
    return pl.pallas_call(
        functools.partial(_vq_body, nsteps),
        grid=(nsteps,),
        in_specs=[
            pl.BlockSpec((BL, DIM), lambda i: (i, 0)),
            pl.BlockSpec((N_CODES, DIM), lambda i: (0, 0)),
        ],
        out_specs=[
            pl.BlockSpec((BL, DIM), lambda i: (i, 0)),
            pl.BlockSpec((BL, N_STAGES), lambda i: (i, 0)),
            pl.BlockSpec((1, 1), lambda i: (0, 0)),
            pl.BlockSpec((1, 1), lambda i: (0, 0)),
        ],
        out_shape=[
            jax.ShapeDtypeStruct((ROWS, DIM), jnp.float32),
            jax.ShapeDtypeStruct((ROWS, N_STAGES), jnp.int32),
            jax.ShapeDtypeStruct((1, 1), jnp.float32),
            jax.ShapeDtypeStruct((1, 1), jnp.float32),
        ],
        scratch_shapes=[pltpu.VMEM((1, N_CODES), jnp.float32)],
    )(z_flat, weight)


_SC_INFO = plsc.get_sparse_core_info()
_NC, _NS = _SC_INFO.num_cores, _SC_INFO.num_subcores
_NW = _NC * _NS
_TOT = N_STAGES * ROWS
_RPW = _TOT // _NW
_GRP = 16
_NGRP = _RPW // _GRP


def _enc_sc(idx_hbm, out_hbm, idx_v, buf0, buf1, sem0, sem1):
    wid = lax.axis_index("s") * _NC + lax.axis_index("c")
    base = wid * _RPW
    pltpu.sync_copy(idx_hbm.at[pl.ds(base * 1, _RPW)], idx_v)

    zeros16 = jnp.zeros((_GRP,), jnp.float32)
    ones16 = jnp.ones((_GRP,), jnp.float32)
    lane = lax.iota(jnp.int32, _GRP)
    bufs = (buf0, buf1)
    sems = (sem0, sem1)

    def _zero(i, c):
        for r in range(_GRP):
            buf0[r, pl.ds(i * _GRP, _GRP)] = zeros16
            buf1[r, pl.ds(i * _GRP, _GRP)] = zeros16
        return c
    lax.fori_loop(0, N_CODES // _GRP, _zero, 0)

    def _offs(g):
        return idx_v[pl.ds(g * _GRP, _GRP)]

    def _fire(b, g):
        plsc.store_scatter(bufs[b], [lane, _offs(g)], ones16)
        pltpu.async_copy(
            bufs[b],
            out_hbm.at[pl.ds(base + g * _GRP, _GRP)],
            sems[b])

    _fire(0, 0)
    _fire(1, 1)

    def _main(gg, c):
        for b in range(2):
            g = gg * 2 + b
            pltpu.make_async_copy(
                bufs[b], out_hbm.at[pl.ds(0, _GRP)], sems[b]).wait()
            plsc.store_scatter(bufs[b], [lane, _offs(g - 2)], zeros16)
            _fire(b, g)
        return c
    lax.fori_loop(1, _NGRP // 2, _main, 0)

    for b in range(2):
        pltpu.make_async_copy(
            bufs[b], out_hbm.at[pl.ds(0, _GRP)], sems[b]).wait()


_enc_kernel = functools.partial(
    pl.kernel,
    mesh=plsc.VectorSubcoreMesh(core_axis_name="c", subcore_axis_name="s"),
    out_type=jax.ShapeDtypeStruct((_TOT, N_CODES), jnp.float32),
    scratch_types=[
        pltpu.VMEM((_RPW,), jnp.int32),
        pltpu.VMEM((_GRP, N_CODES), jnp.float32),
        pltpu.VMEM((_GRP, N_CODES), jnp.float32),
        pltpu.SemaphoreType.DMA,
        pltpu.SemaphoreType.DMA,
    ],
    compiler_params=pltpu.CompilerParams(use_tc_tiling_on_sc=False,
                                         needs_layout_passes=False),
)(_enc_sc)


@jax.jit
def kernel(z, weight):
    b, c, h, w = z.shape
    z_flat = jnp.transpose(z, (0, 2, 3, 1)).reshape(ROWS, DIM)

    zq_flat, idx, loss, ppl = _tc_call(z_flat, weight)

    idx_sm = jnp.transpose(idx).reshape(_TOT)
    encodings_cat = _enc_kernel(idx_sm)

    z_q = jnp.transpose(zq_flat.reshape(b, h, w, DIM), (0, 3, 1, 2))
    indices_stack = jnp.transpose(idx.reshape(b, h, w, N_STAGES), (0, 3, 1, 2))
    return (z_q, loss[0, 0], ppl[0, 0], encodings_cat, indices_stack)


# speedup vs baseline: 2.4852x; 2.4852x over previous
import functools

import jax
import jax.numpy as jnp
from jax import lax
from jax.experimental import pallas as pl
from jax.experimental.pallas import tpu as pltpu

N_CODES = 1024
DIM = 256
N_STAGES = 4
BETA_C = 0.25
BL = 512
NH = 2
H = BL // NH
ROWS = 8192


def _vq_body(nsteps, z_ref, w_ref, zq_ref, enc_ref, idx_ref, loss_ref,
             ppl_ref, cnt_scr):
    i = pl.program_id(0)

    @pl.when(i == 0)
    def _init():
        loss_ref[...] = jnp.zeros_like(loss_ref)
        cnt_scr[...] = jnp.zeros_like(cnt_scr)

    w = w_ref[...]
    wsq = lax.dot_general(jnp.ones((1, DIM), jnp.float32), w * w,
                          (((1,), (1,)), ((), ())),
                          preferred_element_type=jnp.float32,
                          precision=lax.Precision.HIGHEST)
    w1 = w.astype(jnp.bfloat16)
    w2 = (w - w1.astype(jnp.float32)).astype(jnp.bfloat16)
    w3 = (w - w1.astype(jnp.float32) - w2.astype(jnp.float32)).astype(jnp.bfloat16)

    iota = lax.broadcasted_iota(jnp.int32, (H, N_CODES), 1)
    residual = [z_ref[pl.ds(h * H, H), :] for h in range(NH)]
    qsum = [jnp.zeros((H, DIM), jnp.float32) for _ in range(NH)]
    lsum = jnp.zeros((1, 1), jnp.float32)
    cnt = jnp.zeros((1, N_CODES), jnp.float32)
    idx_cols = [[] for _ in range(NH)]
    for q in range(N_STAGES):
        for h in range(NH):
            s = lax.dot_general(residual[h], w, (((1,), (1,)), ((), ())),
                                preferred_element_type=jnp.float32)
            rsq = jnp.sum(residual[h] * residual[h], axis=1, keepdims=True)
            dist = (rsq + wsq) - 2.0 * s
            dmin = jnp.min(dist, axis=1, keepdims=True)
            idxm = jnp.min(jnp.where(dist <= dmin, iota, N_CODES), axis=1,
                           keepdims=True)
            oh = (iota == idxm).astype(jnp.float32)
            enc_ref[q, pl.ds(h * H, H), :] = oh
            cnt = cnt + jnp.sum(oh, axis=0, keepdims=True)
            ohb = oh.astype(jnp.bfloat16)
            zqd = (lax.dot_general(ohb, w1, (((1,), (0,)), ((), ())),
                                   preferred_element_type=jnp.float32)
                   + lax.dot_general(ohb, w2, (((1,), (0,)), ((), ())),
                                     preferred_element_type=jnp.float32)
                   + lax.dot_general(ohb, w3, (((1,), (0,)), ((), ())),
                                     preferred_element_type=jnp.float32))
            qsum[h] = qsum[h] + zqd
            residual[h] = residual[h] - zqd
            r2 = jnp.sum(residual[h] * residual[h], axis=1, keepdims=True)
            lsum = lsum + jnp.sum(r2, axis=0, keepdims=True)
            idx_cols[h].append(idxm)

    for h in range(NH):
        zq_ref[pl.ds(h * H, H), :] = qsum[h]
        idx_ref[pl.ds(h * H, H), :] = jnp.concatenate(idx_cols[h], axis=1)
    loss_ref[...] += lsum
    cnt_scr[...] += cnt

    @pl.when(i == nsteps - 1)
    def _fini():
        loss_ref[...] = loss_ref[...] * (BETA_C / (nsteps * BL * DIM))
        avg = cnt_scr[...] * (1.0 / (nsteps * BL * N_STAGES))
        ent = jnp.sum(avg * jnp.log(avg + 1e-10), axis=1, keepdims=True)
        ppl_ref[...] = jnp.exp(-ent)


@jax.jit
def kernel(z, weight):
    b, c, h, w = z.shape
    nsteps = ROWS // BL
    z_flat = jnp.transpose(z, (0, 2, 3, 1)).reshape(ROWS, DIM)

    zq_flat, enc, idx, loss, ppl = pl.pallas_call(
        functools.partial(_vq_body, nsteps),
        grid=(nsteps,),
        in_specs=[
            pl.BlockSpec((BL, DIM), lambda i: (i, 0)),
            pl.BlockSpec((N_CODES, DIM), lambda i: (0, 0)),
        ],
        out_specs=[
            pl.BlockSpec((BL, DIM), lambda i: (i, 0)),
            pl.BlockSpec((N_STAGES, BL, N_CODES), lambda i: (0, i, 0)),
            pl.BlockSpec((BL, N_STAGES), lambda i: (i, 0)),
            pl.BlockSpec((1, 1), lambda i: (0, 0)),
            pl.BlockSpec((1, 1), lambda i: (0, 0)),
        ],
        out_shape=[
            jax.ShapeDtypeStruct((ROWS, DIM), jnp.float32),
            jax.ShapeDtypeStruct((N_STAGES, ROWS, N_CODES), jnp.float32),
            jax.ShapeDtypeStruct((ROWS, N_STAGES), jnp.int32),
            jax.ShapeDtypeStruct((1, 1), jnp.float32),
            jax.ShapeDtypeStruct((1, 1), jnp.float32),
        ],
        scratch_shapes=[pltpu.VMEM((1, N_CODES), jnp.float32)],
    )(z_flat, weight)

    z_q = jnp.transpose(zq_flat.reshape(b, h, w, DIM), (0, 3, 1, 2))
    encodings_cat = enc.reshape(N_STAGES * ROWS, N_CODES)
    indices_stack = jnp.transpose(idx.reshape(b, h, w, N_STAGES), (0, 3, 1, 2))
    return (z_q, loss[0, 0], ppl[0, 0], encodings_cat, indices_stack)
